# baseline (speedup 1.0000x reference)
import functools

import jax
import jax.numpy as jnp
from jax import lax
from jax.experimental import pallas as pl
from jax.experimental.pallas import tpu as pltpu
from jax.experimental.pallas import tpu_sc as plsc

_BS = 1024

_NC = 2
_NS = 16
_NW = _NC * _NS
_R = 16


def _tc_add_kernel(x_ref, pos_ref, o_ref):
    i = pl.program_id(0)
    o_ref[0] = x_ref[0] + pos_ref[pl.ds(i * _BS, _BS), :]


def _tc_add(x, pos):
    batch, seq_len, d_model = x.shape
    grid = (seq_len // _BS, batch)
    return pl.pallas_call(
        _tc_add_kernel,
        grid=grid,
        in_specs=[
            pl.BlockSpec((1, _BS, d_model), lambda i, j: (j, i, 0)),
            pl.BlockSpec((seq_len, d_model), lambda i, j: (0, 0)),
        ],
        out_specs=pl.BlockSpec((1, _BS, d_model), lambda i, j: (j, i, 0)),
        out_shape=jax.ShapeDtypeStruct(x.shape, x.dtype),
        compiler_params=pltpu.CompilerParams(
            vmem_limit_bytes=64 * 1024 * 1024),
    )(x, pos)


def _sc_add(x, pos):
    batch, seq_len, d_model = x.shape
    spw = seq_len // _NW
    n_chunks = spw // _R
    chunk_w = _R * d_model
    n_tasks = n_chunks * batch

    mesh = plsc.VectorSubcoreMesh(
        core_axis_name="c", subcore_axis_name="s",
        num_cores=_NC, num_subcores=_NS)

    @functools.partial(
        pl.kernel,
        out_type=jax.ShapeDtypeStruct((batch * seq_len * d_model,), jnp.float32),
        mesh=mesh,
        scratch_types=(
            [pltpu.VMEM((chunk_w,), jnp.float32)] * 6
            + [pltpu.SemaphoreType.DMA] * 6
        ),
    )
    def sc_kernel(x_hbm, pos_hbm, o_hbm,
                  xb0, xb1, pb0, pb1, ob0, ob1,
                  is0, is1, ps0, ps1, os0, os1):
        wid = lax.axis_index("s") * _NC + lax.axis_index("c")
        seq_base = wid * spw
        xb, pb, ob = [xb0, xb1], [pb0, pb1], [ob0, ob1]
        isem, psem, osem = [is0, is1], [ps0, ps1], [os0, os1]

        def x_slice(t):
            s, b = divmod(t, batch)
            off = (b * seq_len + seq_base + s * _R) * d_model
            return pl.ds(off, chunk_w)

        def p_slice(s):
            return pl.ds((seq_base + s * _R) * d_model, chunk_w)

        pltpu.async_copy(pos_hbm.at[p_slice(0)], pb[0], psem[0])
        if n_chunks > 1:
            pltpu.async_copy(pos_hbm.at[p_slice(1)], pb[1], psem[1])
        pltpu.async_copy(x_hbm.at[x_slice(0)], xb[0], isem[0])
        if n_tasks > 1:
            pltpu.async_copy(x_hbm.at[x_slice(1)], xb[1], isem[1])

        for t in range(n_tasks):
            i = t % 2
            s, b = divmod(t, batch)
            pltpu.make_async_copy(x_hbm.at[x_slice(t)], xb[i], isem[i]).wait()
            if b == 0:
                pltpu.make_async_copy(
                    pos_hbm.at[p_slice(s)], pb[s % 2], psem[s % 2]).wait()
            if t >= 2:
                pltpu.make_async_copy(
                    ob[i], o_hbm.at[x_slice(t - 2)], osem[i]).wait()

            xbi, pbi, obi = xb[i], pb[s % 2], ob[i]

            @plsc.parallel_loop(0, chunk_w, 16, unroll=8)
            def _add(off):
                sl = pl.ds(off, 16)
                obi[sl] = xbi[sl] + pbi[sl]

            if t + 2 < n_tasks:
                pltpu.async_copy(x_hbm.at[x_slice(t + 2)], xb[i], isem[i])
            if b == batch - 1 and s + 2 < n_chunks:
                pltpu.async_copy(
                    pos_hbm.at[p_slice(s + 2)], pb[s % 2], psem[s % 2])
            pltpu.async_copy(ob[i], o_hbm.at[x_slice(t)], osem[i])

        for t in range(max(0, n_tasks - 2), n_tasks):
            i = t % 2
            pltpu.make_async_copy(ob[i], o_hbm.at[x_slice(t)], osem[i]).wait()

    out = sc_kernel(x.reshape(-1), pos.reshape(-1))
    return out.reshape(x.shape)


def kernel(x, pos_table):
    seq_len = x.shape[1]
    pos = pos_table if seq_len == pos_table.shape[0] else pos_table[:seq_len]
    return _tc_add(x, pos)


# speedup vs baseline: 1.0036x; 1.0036x over previous
import functools

import jax
import jax.numpy as jnp
from jax import lax
from jax.experimental import pallas as pl
from jax.experimental.pallas import tpu as pltpu
from jax.experimental.pallas import tpu_sc as plsc

_BS = 2048

_NC = 2
_NS = 16
_NW = _NC * _NS
_R = 16


def _tc_add_kernel(x_ref, pos_hbm, o_ref, pbuf, sem):
    i = pl.program_id(0)
    j = pl.program_id(1)
    ni = pl.num_programs(0)

    @pl.when((i == 0) & (j == 0))
    def _():
        pltpu.make_async_copy(
            pos_hbm.at[pl.ds(0, _BS), :], pbuf.at[0], sem).start()

    @pl.when(j == 0)
    def _():
        pltpu.make_async_copy(
            pos_hbm.at[pl.ds(i * _BS, _BS), :], pbuf.at[i % 2], sem).wait()

    @pl.when((j == 0) & (i + 1 < ni))
    def _():
        pltpu.make_async_copy(
            pos_hbm.at[pl.ds((i + 1) * _BS, _BS), :],
            pbuf.at[(i + 1) % 2], sem).start()

    o_ref[0] = x_ref[0] + pbuf[i % 2]


def _tc_add(x, pos):
    batch, seq_len, d_model = x.shape
    grid = (seq_len // _BS, batch)
    return pl.pallas_call(
        _tc_add_kernel,
        grid=grid,
        in_specs=[
            pl.BlockSpec((1, _BS, d_model), lambda i, j: (j, i, 0)),
            pl.BlockSpec(memory_space=pl.ANY),
        ],
        out_specs=pl.BlockSpec((1, _BS, d_model), lambda i, j: (j, i, 0)),
        out_shape=jax.ShapeDtypeStruct(x.shape, x.dtype),
        scratch_shapes=[
            pltpu.VMEM((2, _BS, d_model), jnp.float32),
            pltpu.SemaphoreType.DMA,
        ],
        compiler_params=pltpu.CompilerParams(
            vmem_limit_bytes=64 * 1024 * 1024),
    )(x, pos)


def _sc_add(x, pos):
    batch, seq_len, d_model = x.shape
    spw = seq_len // _NW
    n_chunks = spw // _R
    chunk_w = _R * d_model
    n_tasks = n_chunks * batch

    mesh = plsc.VectorSubcoreMesh(
        core_axis_name="c", subcore_axis_name="s",
        num_cores=_NC, num_subcores=_NS)

    @functools.partial(
        pl.kernel,
        out_type=jax.ShapeDtypeStruct((batch * seq_len * d_model,), jnp.float32),
        mesh=mesh,
        scratch_types=(
            [pltpu.VMEM((chunk_w,), jnp.float32)] * 6
            + [pltpu.SemaphoreType.DMA] * 6
        ),
    )
    def sc_kernel(x_hbm, pos_hbm, o_hbm,
                  xb0, xb1, pb0, pb1, ob0, ob1,
                  is0, is1, ps0, ps1, os0, os1):
        wid = lax.axis_index("s") * _NC + lax.axis_index("c")
        seq_base = wid * spw
        xb, pb, ob = [xb0, xb1], [pb0, pb1], [ob0, ob1]
        isem, psem, osem = [is0, is1], [ps0, ps1], [os0, os1]

        def x_slice(t):
            s, b = divmod(t, batch)
            off = (b * seq_len + seq_base + s * _R) * d_model
            return pl.ds(off, chunk_w)

        def p_slice(s):
            return pl.ds((seq_base + s * _R) * d_model, chunk_w)

        pltpu.async_copy(pos_hbm.at[p_slice(0)], pb[0], psem[0])
        if n_chunks > 1:
            pltpu.async_copy(pos_hbm.at[p_slice(1)], pb[1], psem[1])
        pltpu.async_copy(x_hbm.at[x_slice(0)], xb[0], isem[0])
        if n_tasks > 1:
            pltpu.async_copy(x_hbm.at[x_slice(1)], xb[1], isem[1])

        for t in range(n_tasks):
            i = t % 2
            s, b = divmod(t, batch)
            pltpu.make_async_copy(x_hbm.at[x_slice(t)], xb[i], isem[i]).wait()
            if b == 0:
                pltpu.make_async_copy(
                    pos_hbm.at[p_slice(s)], pb[s % 2], psem[s % 2]).wait()
            if t >= 2:
                pltpu.make_async_copy(
                    ob[i], o_hbm.at[x_slice(t - 2)], osem[i]).wait()

            xbi, pbi, obi = xb[i], pb[s % 2], ob[i]

            @plsc.parallel_loop(0, chunk_w, 16, unroll=8)
            def _add(off):
                sl = pl.ds(off, 16)
                obi[sl] = xbi[sl] + pbi[sl]

            if t + 2 < n_tasks:
                pltpu.async_copy(x_hbm.at[x_slice(t + 2)], xb[i], isem[i])
            if b == batch - 1 and s + 2 < n_chunks:
                pltpu.async_copy(
                    pos_hbm.at[p_slice(s + 2)], pb[s % 2], psem[s % 2])
            pltpu.async_copy(ob[i], o_hbm.at[x_slice(t)], osem[i])

        for t in range(max(0, n_tasks - 2), n_tasks):
            i = t % 2
            pltpu.make_async_copy(ob[i], o_hbm.at[x_slice(t)], osem[i]).wait()

    out = sc_kernel(x.reshape(-1), pos.reshape(-1))
    return out.reshape(x.shape)


def kernel(x, pos_table):
    seq_len = x.shape[1]
    pos = pos_table if seq_len == pos_table.shape[0] else pos_table[:seq_len]
    return _tc_add(x, pos)
